# baseline (speedup 1.0000x reference)
import jax
import jax.numpy as jnp
from jax.experimental import pallas as pl
from jax.experimental.pallas import tpu as pltpu

_ALPHA = 1.5
_NEG = -1e20
_LOG2E = 1.4426950408889634

_Q0 = 0.98402748
_Q1 = -0.40917639
_Q2 = 0.14045614
_Q3 = -0.02234705


def _log1p(t):
    return t * (_Q0 + t * (_Q1 + t * (_Q2 + t * _Q3)))


def _lse2(a, b):
    m = jnp.maximum(a, b)
    t = jnp.exp2((jnp.minimum(a, b) - m) * _LOG2E)
    return m + _log1p(t)


def _dp_kernel(w_ref, out_ref, rowbuf_scr, state_scr):
    c = pl.program_id(0)
    batch, na, nb = w_ref.shape
    rows = 32
    nq = 4
    q = batch // nq
    quarters = [slice(k * q, (k + 1) * q) for k in range(nq)]
    is_first = c == 0
    rd = (c - 1) % 2
    wr = c % 2

    rb = rowbuf_scr[rd]
    for h, sl in enumerate(quarters):
        out_ref[sl, :, 1 : nb + 1] = jnp.swapaxes(rb[:, sl, :], 0, 1)

    slot = jax.lax.broadcasted_iota(jnp.int32, (batch, rows, 1), 1)
    col0 = jnp.where((c == 1) & (slot == 0), 0.0, _NEG).astype(jnp.float32)
    out_ref[:, :, 0:1] = col0

    base = jnp.minimum(c, na // rows - 1) * rows
    w_chunk = w_ref[:, pl.ds(base, rows), :]
    aw = _ALPHA * jnp.swapaxes(w_chunk, 0, 1)

    neg_col = jnp.full((q, 1), _NEG, dtype=jnp.float32)
    zero_col = jnp.zeros((q, 1), dtype=jnp.float32)
    row0_int = jnp.full((q, nb), _NEG, jnp.float32)
    border0 = jnp.where(is_first, zero_col, neg_col)

    for h, sl in enumerate(quarters):
        s = jnp.where(is_first, row0_int, state_scr[sl, :])
        for p in range(rows // 2):
            a_row = aw[2 * p, sl, :]
            b_row = aw[2 * p + 1, sl, :]
            a1 = jnp.concatenate([neg_col, a_row[:, :-1]], axis=1)
            cc = _lse2(a_row, a1)
            border = border0 if p == 0 else neg_col
            sh1 = jnp.concatenate([border, s[:, :-1]], axis=1)
            sh2 = jnp.concatenate([neg_col, border, s[:, :-2]], axis=1)
            u1 = s + a_row
            u2 = sh1 + cc
            u3 = sh2 + a1
            m = jnp.maximum(jnp.maximum(u1, u2), u3)
            t = (jnp.exp2((u1 - m) * _LOG2E) + jnp.exp2((u2 - m) * _LOG2E)
                 + jnp.exp2((u3 - m) * _LOG2E)) - 1.0
            z = b_row + (m + _log1p(t))
            y = a_row + _lse2(s, sh1)
            rowbuf_scr[wr, 2 * p + 1, sl, :] = y
            if p < rows // 2 - 1:
                rowbuf_scr[wr, 2 * p + 2, sl, :] = z
            else:
                rowbuf_scr[rd, 0, sl, :] = z
            s = z
        state_scr[sl, :] = s

    @pl.when(is_first)
    def _():
        rowbuf_scr[wr, 0] = jnp.full((batch, nb), _NEG, jnp.float32)


def kernel(W):
    batch, na, nb = W.shape
    chunk = 32
    n_steps = (na + 1 + chunk - 1) // chunk + 1
    del chunk

    return pl.pallas_call(
        _dp_kernel,
        grid=(n_steps,),
        in_specs=[pl.BlockSpec((batch, na, nb), lambda c: (0, 0, 0))],
        out_specs=pl.BlockSpec(
            (batch, 32, nb + 1),
            lambda c: (0, jnp.maximum(c - 1, 0), 0),
        ),
        out_shape=jax.ShapeDtypeStruct((batch, na + 1, nb + 1), W.dtype),
        scratch_shapes=[
            pltpu.VMEM((2, 32, batch, nb), jnp.float32),
            pltpu.VMEM((batch, nb), jnp.float32),
        ],
    )(W)


# speedup vs baseline: 1.0476x; 1.0476x over previous
import jax
import jax.numpy as jnp
from jax.experimental import pallas as pl
from jax.experimental.pallas import tpu as pltpu

_ALPHA = 1.5
_NEG = -1e20
_LOG2E = 1.4426950408889634

_Q0 = 0.98402748
_Q1 = -0.40917639
_Q2 = 0.14045614
_Q3 = -0.02234705


def _log1p(t):
    return t * (_Q0 + t * (_Q1 + t * (_Q2 + t * _Q3)))


def _lse2(a, b):
    m = jnp.maximum(a, b)
    t = jnp.exp2((jnp.minimum(a, b) - m) * _LOG2E)
    return m + _log1p(t)


def _dp_kernel(w_ref, out_ref, rowbuf_scr, state_scr):
    c = pl.program_id(0)
    batch, rows, nb = w_ref.shape
    nq = 4
    q = batch // nq
    quarters = [slice(k * q, (k + 1) * q) for k in range(nq)]
    is_first = c == 0
    rd = (c - 1) % 2
    wr = c % 2

    rb = rowbuf_scr[rd]
    for h, sl in enumerate(quarters):
        out_ref[sl, :, 1 : nb + 1] = jnp.swapaxes(rb[:, sl, :], 0, 1)

    slot = jax.lax.broadcasted_iota(jnp.int32, (batch, rows, 1), 1)
    col0 = jnp.where((c == 1) & (slot == 0), 0.0, _NEG).astype(jnp.float32)
    out_ref[:, :, 0:1] = col0

    aw = _ALPHA * jnp.swapaxes(w_ref[...], 0, 1)

    neg_col = jnp.full((q, 1), _NEG, dtype=jnp.float32)
    zero_col = jnp.zeros((q, 1), dtype=jnp.float32)
    row0_int = jnp.full((q, nb), _NEG, jnp.float32)
    border0 = jnp.where(is_first, zero_col, neg_col)

    for h, sl in enumerate(quarters):
        s = jnp.where(is_first, row0_int, state_scr[sl, :])
        for p in range(rows // 2):
            a_row = aw[2 * p, sl, :]
            b_row = aw[2 * p + 1, sl, :]
            a1 = jnp.concatenate([neg_col, a_row[:, :-1]], axis=1)
            cc = _lse2(a_row, a1)
            border = border0 if p == 0 else neg_col
            sh1 = jnp.concatenate([border, s[:, :-1]], axis=1)
            sh2 = jnp.concatenate([neg_col, border, s[:, :-2]], axis=1)
            u1 = s + a_row
            u2 = sh1 + cc
            u3 = sh2 + a1
            m = jnp.maximum(jnp.maximum(u1, u2), u3)
            t = (jnp.exp2((u1 - m) * _LOG2E) + jnp.exp2((u2 - m) * _LOG2E)
                 + jnp.exp2((u3 - m) * _LOG2E)) - 1.0
            z = b_row + (m + _log1p(t))
            y = a_row + _lse2(s, sh1)
            rowbuf_scr[wr, 2 * p + 1, sl, :] = y
            if p < rows // 2 - 1:
                rowbuf_scr[wr, 2 * p + 2, sl, :] = z
            else:
                rowbuf_scr[rd, 0, sl, :] = z
            s = z
        state_scr[sl, :] = s

    @pl.when(is_first)
    def _():
        rowbuf_scr[wr, 0] = jnp.full((batch, nb), _NEG, jnp.float32)


def kernel(W):
    batch, na, nb = W.shape
    chunk = 32
    n_steps = (na + 1 + chunk - 1) // chunk + 1
    w_blocks = na // chunk

    return pl.pallas_call(
        _dp_kernel,
        grid=(n_steps,),
        in_specs=[
            pl.BlockSpec(
                (batch, 32, nb),
                lambda c: (0, jnp.minimum(c, w_blocks - 1), 0),
            )
        ],
        out_specs=pl.BlockSpec(
            (batch, 32, nb + 1),
            lambda c: (0, jnp.maximum(c - 1, 0), 0),
        ),
        out_shape=jax.ShapeDtypeStruct((batch, na + 1, nb + 1), W.dtype),
        scratch_shapes=[
            pltpu.VMEM((2, 32, batch, nb), jnp.float32),
            pltpu.VMEM((batch, nb), jnp.float32),
        ],
    )(W)
